# baseline (speedup 1.0000x reference)
import jax
import jax.numpy as jnp
from jax import lax
from jax.experimental import pallas as pl
from jax.experimental.pallas import tpu as pltpu
from jax.experimental.pallas import tpu_sc as plsc

SIZE = 1024
DEPTH = 16
NPIX = 4 * 512 * 512
LEVEL_W = [SIZE >> l for l in range(4)]
LEVEL_BASE = [0, 1048576, 1310720, 1376256]

P = 128
G = P // 16
NK = 16
NROW = NK * P
CHUNK = 128
NCH = NROW // CHUNK
CPB = 262144
CH_STRIDE = 262144
B_STRIDE = 16 * 262144


def _sc_body(uv_hbm, t0_hbm, t1_hbm, t2_hbm, t3_hbm, out_hbm,
             uv_v, idx_v, w_v, rows_v, out_v, usem, gsem, osem):
    info = plsc.get_sparse_core_info()
    nc = info.num_cores
    wid = lax.axis_index("s") * nc + lax.axis_index("c")
    npix_w = NPIX // (nc * info.num_subcores)
    nblk = npix_w // P
    pixw0 = wid * npix_w

    iota = lax.iota(jnp.int32, 16)

    def fire_uv(blk, pr):
        pix0 = pixw0 + blk * P
        pltpu.async_copy(uv_hbm.at[pl.ds(2 * pix0, 2 * P)], uv_v.at[pr], usem[pr])

    def compute_block(blk, pr):
        pltpu.make_async_copy(
            uv_hbm.at[pl.ds(0, 2 * P)], uv_v.at[pr], usem[pr]).wait()

        def idx_body(g, _):
            p0 = g * 16
            ux = plsc.load_gather(uv_v, [jnp.broadcast_to(pr, (16,)),
                                         (p0 + iota) * 2])
            uy = plsc.load_gather(uv_v, [jnp.broadcast_to(pr, (16,)),
                                         (p0 + iota) * 2 + 1])
            gx = 2.0 * ux - 1.0
            gy = 2.0 * uy - 1.0
            for l in range(4):
                w = LEVEL_W[l]
                base = LEVEL_BASE[l]
                ix = ((gx + 1.0) * w - 1.0) / 2.0
                iy = ((gy + 1.0) * w - 1.0) / 2.0
                tx = ix.astype(jnp.int32)
                x0 = tx - jnp.where(tx.astype(jnp.float32) > ix, 1, 0)
                ty = iy.astype(jnp.int32)
                y0 = ty - jnp.where(ty.astype(jnp.float32) > iy, 1, 0)
                wx1 = ix - x0.astype(jnp.float32)
                wx0 = 1.0 - wx1
                wy1 = iy - y0.astype(jnp.float32)
                wy0 = 1.0 - wy1
                ax0 = wx0 * jnp.where(x0 >= 0, 1.0, 0.0)
                ax1 = wx1 * jnp.where(x0 <= w - 2, 1.0, 0.0)
                ay0 = wy0 * jnp.where(y0 >= 0, 1.0, 0.0)
                ay1 = wy1 * jnp.where(y0 <= w - 2, 1.0, 0.0)
                xc0 = jnp.maximum(x0, 0)
                xc1 = jnp.minimum(x0 + 1, w - 1)
                yc0 = jnp.maximum(y0, 0)
                yc1 = jnp.minimum(y0 + 1, w - 1)
                del base
                r0 = yc0 * w + xc0
                r1 = yc0 * w + xc1
                r2 = yc1 * w + xc0
                r3 = yc1 * w + xc1
                for ci, (rr, ww) in enumerate(
                        ((r0, ax0 * ay0), (r1, ax1 * ay0),
                         (r2, ax0 * ay1), (r3, ax1 * ay1))):
                    k = l * 4 + ci
                    idx_v[pr, pl.ds(k * P + p0, 16)] = rr
                    w_v[pr, pl.ds(k * P + p0, 16)] = ww
            return 0

        lax.fori_loop(0, G, idx_body, 0, unroll=False)
        tables = (t0_hbm, t1_hbm, t2_hbm, t3_hbm)
        for j in range(NCH):
            pltpu.async_copy(
                tables[j // 4].at[
                    idx_v.at[pr, pl.ds(j * CHUNK, CHUNK)]],
                rows_v.at[pr, pl.ds(j * CHUNK, CHUNK)], gsem[pr])
        @pl.when(blk + 2 < nblk)
        def _():
            fire_uv(blk + 2, pr)

    def finish_block(blk, pr, drain_out):
        if drain_out:
            @pl.when(blk >= 2)
            def _():
                pltpu.make_async_copy(
                    uv_hbm.at[pl.ds(0, 16 * P)], out_v.at[pr],
                    osem[pr]).wait()
        pltpu.make_async_copy(
            t0_hbm.at[pl.ds(0, NROW)], rows_v.at[pr], gsem[pr]).wait()

        def acc_body(g, _):
            p0 = g * 16
            accs = [jnp.zeros((16,), jnp.float32) for _ in range(16)]
            prv = jnp.broadcast_to(pr, (16,))
            for k in range(NK):
                wk = w_v[pr, pl.ds(k * P + p0, 16)]
                rid = k * P + p0 + iota
                for c in range(16):
                    r = plsc.load_gather(
                        rows_v, [prv, rid, jnp.full((16,), c, jnp.int32)])
                    accs[c] = accs[c] + wk * r
            for c in range(16):
                out_v[pr, pl.ds(c * P + p0, 16)] = accs[c]
            return 0

        lax.fori_loop(0, G, acc_body, 0, unroll=False)

        pix0 = pixw0 + blk * P
        b = pix0 // CPB
        pib = pix0 % CPB
        obase = b * B_STRIDE + pib
        for c in range(16):
            pltpu.async_copy(
                out_v.at[pr, pl.ds(c * P, P)],
                out_hbm.at[pl.ds(obase + c * CH_STRIDE, P)], osem[pr])

    fire_uv(0, 0)
    fire_uv(1, 1)
    compute_block(0, 0)

    def pipe_body(t, _):
        blk = 2 * t
        compute_block(blk + 1, 1)
        finish_block(blk, 0, drain_out=True)

        @pl.when(blk + 2 < nblk)
        def _():
            compute_block(blk + 2, 0)

        finish_block(blk + 1, 1, drain_out=True)
        return 0

    lax.fori_loop(0, nblk // 2, pipe_body, 0, unroll=False)

    for pr in range(2):
        pltpu.make_async_copy(
            uv_hbm.at[pl.ds(0, 16 * P)], out_v.at[pr], osem[pr]).wait()


@jax.jit
def _run(uv, t0, t1, t2, t3):
    mesh = plsc.VectorSubcoreMesh(core_axis_name="c", subcore_axis_name="s")
    f = pl.kernel(
        _sc_body,
        out_type=jax.ShapeDtypeStruct((NPIX * 16,), jnp.float32),
        mesh=mesh,
        compiler_params=pltpu.CompilerParams(
            needs_layout_passes=False, use_tc_tiling_on_sc=False),
        scratch_types=[
            pltpu.VMEM((2, 2 * P), jnp.float32),
            pltpu.VMEM((2, NROW), jnp.int32),
            pltpu.VMEM((2, NROW), jnp.float32),
            pltpu.VMEM((2, NROW, 16), jnp.float32),
            pltpu.VMEM((2, 16 * P), jnp.float32),
            [pltpu.SemaphoreType.DMA, pltpu.SemaphoreType.DMA],
            [pltpu.SemaphoreType.DMA, pltpu.SemaphoreType.DMA],
            [pltpu.SemaphoreType.DMA, pltpu.SemaphoreType.DMA],
        ],
    )
    return f(uv, t0, t1, t2, t3)


def kernel(input, mipmap_0, mipmap_1, mipmap_2, mipmap_3):
    uv = input.reshape(-1)
    tables = [m[0].transpose(1, 2, 0).reshape(-1, DEPTH)
              for m in (mipmap_0, mipmap_1, mipmap_2, mipmap_3)]
    out = _run(uv, *tables)
    return out.reshape(4, DEPTH, 512, 512)


# speedup vs baseline: 2.2353x; 2.2353x over previous
import jax
import jax.numpy as jnp
from jax import lax
from jax.experimental import pallas as pl
from jax.experimental.pallas import tpu as pltpu
from jax.experimental.pallas import tpu_sc as plsc

SIZE = 1024
DEPTH = 16
NPIX = 4 * 512 * 512
LEVEL_W = [SIZE >> l for l in range(4)]
LEVEL_BASE = [0, 1048576, 1310720, 1376256]
TABV = 1392640

P = 128
G = P // 16
NK = 8
NROW = NK * P
CHUNK = 128
NCH = NROW // CHUNK
CPB = 262144
CH_STRIDE = 262144
B_STRIDE = 16 * 262144
T = 512


def _sc_body(u_hbm, v_hbm, m0_hbm, m1_hbm, m2_hbm, m3_hbm,
             out_hbm, tab_hbm,
             u_v, v_v, idx_v, w_v, rows_v, out_v, cb_v, row_v,
             usem, gsem, osem, csem, tsem):
    info = plsc.get_sparse_core_info()
    nc = info.num_cores
    cid = lax.axis_index("c")
    sid = lax.axis_index("s")
    wid = sid * nc + cid
    npix_w = NPIX // (nc * info.num_subcores)
    nblk = npix_w // P
    pixw0 = wid * npix_w

    iota = lax.iota(jnp.int32, 16)

    def fire_reads(m_hbm, start, par):
        for ch in range(16):
            pltpu.async_copy(m_hbm.at[ch, pl.ds(start, T + 8)],
                             cb_v.at[par, ch, pl.ds(0, T + 8)], csem[par])

    def drain_reads(m_hbm, par):
        for ch in range(16):
            pltpu.make_async_copy(m_hbm.at[ch, pl.ds(0, T + 8)],
                                  cb_v.at[par, ch, pl.ds(0, T + 8)],
                                  csem[par]).wait()

    for l, m_hbm in enumerate((m0_hbm, m1_hbm, m2_hbm, m3_hbm)):
        hw = LEVEL_W[l] * LEVEL_W[l]
        per_tile = hw // 16
        nck = per_tile // T
        tile0 = sid * per_tile
        lvl_base = LEVEL_BASE[l]

        def win_start(ck, nck=nck, tile0=tile0):
            is_final = jnp.logical_and(ck == nck - 1, sid == 15)
            off = jnp.where(is_final, 8, 0)
            return tile0 + ck * T - off, off

        def do_chunk(ck, par, l=l, m_hbm=m_hbm, nck=nck, tile0=tile0,
                     lvl_base=lvl_base):
            t0 = tile0 + ck * T
            _, off = win_start(ck)
            drain_reads(m_hbm, par)

            @pl.when(ck >= 2)
            def _():
                pltpu.make_async_copy(
                    tab_hbm.at[pl.ds(0, T)], row_v.at[par],
                    tsem[par]).wait()

            def grp_body(g, _):
                r0 = g * 16 + off
                pos = g * 16 + iota
                for wd in range(8):
                    a0 = cb_v[par, 2 * wd, pl.ds(r0, 16)]
                    a1 = cb_v[par, 2 * wd + 1, pl.ds(r0, 16)]
                    wa = plsc.bitcast(
                        plsc.pack(a0, a1, format=plsc.PackFormat.INTERLEAVED),
                        jnp.int32)
                    b0 = cb_v[par, 2 * wd, pl.ds(r0 + 1, 16)]
                    b1 = cb_v[par, 2 * wd + 1, pl.ds(r0 + 1, 16)]
                    wb = plsc.bitcast(
                        plsc.pack(b0, b1, format=plsc.PackFormat.INTERLEAVED),
                        jnp.int32)
                    plsc.store_scatter(
                        row_v, [jnp.broadcast_to(par, (16,)), pos,
                                jnp.full((16,), wd, jnp.int32)], wa)
                    plsc.store_scatter(
                        row_v, [jnp.broadcast_to(par, (16,)), pos,
                                jnp.full((16,), 8 + wd, jnp.int32)], wb)
                return 0

            lax.fori_loop(0, T // 16, grp_body, 0, unroll=False)
            pltpu.async_copy(row_v.at[par],
                             tab_hbm.at[pl.ds(lvl_base + t0, T)], tsem[par])

            @pl.when(ck + 2 < nck)
            def _():
                s2, _o = win_start(ck + 2)
                fire_reads(m_hbm, s2, par)

        s0, _ = win_start(0)
        fire_reads(m_hbm, s0, 0)
        s1, _ = win_start(1)
        fire_reads(m_hbm, s1, 1)

        def pair_body(t, _):
            do_chunk(2 * t, 0)
            do_chunk(2 * t + 1, 1)
            return 0

        lax.fori_loop(0, nck // 2, pair_body, 0, unroll=False)
        for par in range(2):
            pltpu.make_async_copy(
                tab_hbm.at[pl.ds(0, T)], row_v.at[par],
                tsem[par]).wait()

    plsc.subcore_barrier()

    def fire_uv(blk, pr):
        pix0 = pixw0 + blk * P
        pltpu.async_copy(u_hbm.at[pl.ds(pix0, P)], u_v.at[pr], usem[pr])
        pltpu.async_copy(v_hbm.at[pl.ds(pix0, P)], v_v.at[pr], usem[pr])

    def compute_block(blk, pr):
        pltpu.make_async_copy(u_hbm.at[pl.ds(0, P)], u_v.at[pr], usem[pr]).wait()
        pltpu.make_async_copy(v_hbm.at[pl.ds(0, P)], v_v.at[pr], usem[pr]).wait()

        def idx_body(g, _):
            p0 = g * 16
            ux = u_v[pr, pl.ds(p0, 16)]
            uy = v_v[pr, pl.ds(p0, 16)]
            gx = 2.0 * ux - 1.0
            gy = 2.0 * uy - 1.0
            for l in range(4):
                w = LEVEL_W[l]
                ix = ((gx + 1.0) * w - 1.0) / 2.0
                iy = ((gy + 1.0) * w - 1.0) / 2.0
                tx = ix.astype(jnp.int32)
                x0 = tx - jnp.where(tx.astype(jnp.float32) > ix, 1, 0)
                ty = iy.astype(jnp.int32)
                y0 = ty - jnp.where(ty.astype(jnp.float32) > iy, 1, 0)
                wx1 = ix - x0.astype(jnp.float32)
                wx0 = 1.0 - wx1
                wy1 = iy - y0.astype(jnp.float32)
                wy0 = 1.0 - wy1
                is_l = x0 < 0
                is_r = x0 > w - 2
                sa = jnp.where(is_l, wx1, jnp.where(is_r, 0.0, wx0))
                sb = jnp.where(is_l, 0.0, jnp.where(is_r, wx0, wx1))
                ay0 = wy0 * jnp.where(y0 >= 0, 1.0, 0.0)
                ay1 = wy1 * jnp.where(y0 <= w - 2, 1.0, 0.0)
                xs = jnp.minimum(jnp.maximum(x0, 0), w - 2)
                yc0 = jnp.maximum(y0, 0)
                yc1 = jnp.minimum(y0 + 1, w - 1)
                base = LEVEL_BASE[l]
                for yr, (yc, ay) in enumerate(((yc0, ay0), (yc1, ay1))):
                    k = l * 2 + yr
                    idx_v[pr, pl.ds(k * P + p0, 16)] = yc * w + (base + xs)
                    w_v[pr, pl.ds((2 * k) * P + p0, 16)] = sa * ay
                    w_v[pr, pl.ds((2 * k + 1) * P + p0, 16)] = sb * ay
            return 0

        lax.fori_loop(0, G, idx_body, 0, unroll=False)
        for j in range(NCH):
            pltpu.async_copy(
                tab_hbm.at[idx_v.at[pr, pl.ds(j * CHUNK, CHUNK)]],
                rows_v.at[pr, pl.ds(j * CHUNK, CHUNK)], gsem[pr])

        @pl.when(blk + 2 < nblk)
        def _():
            fire_uv(blk + 2, pr)

    def finish_block(blk, pr):
        @pl.when(blk >= 2)
        def _():
            pltpu.make_async_copy(
                u_hbm.at[pl.ds(0, 16 * P)], out_v.at[pr], osem[pr]).wait()

        pltpu.make_async_copy(
            tab_hbm.at[pl.ds(0, NROW)], rows_v.at[pr], gsem[pr]).wait()

        mask_hi = jnp.full((16,), -65536, jnp.int32)

        def acc_body(g, _):
            p0 = g * 16
            accs = [jnp.zeros((16,), jnp.float32) for _ in range(16)]
            prv = jnp.broadcast_to(pr, (16,))
            for k in range(NK):
                wa = w_v[pr, pl.ds((2 * k) * P + p0, 16)]
                wb = w_v[pr, pl.ds((2 * k + 1) * P + p0, 16)]
                rid = k * P + p0 + iota
                for wd in range(8):
                    for half, wv in ((0, wa), (1, wb)):
                        vec = plsc.load_gather(
                            rows_v, [prv, rid,
                                     jnp.full((16,), 8 * half + wd, jnp.int32)])
                        ch_even = plsc.bitcast(vec << 16, jnp.float32)
                        ch_odd = plsc.bitcast(vec & mask_hi, jnp.float32)
                        accs[2 * wd] = accs[2 * wd] + wv * ch_even
                        accs[2 * wd + 1] = accs[2 * wd + 1] + wv * ch_odd
            for c in range(16):
                out_v[pr, pl.ds(c * P + p0, 16)] = accs[c]
            return 0

        lax.fori_loop(0, G, acc_body, 0, unroll=False)

        pix0 = pixw0 + blk * P
        b = pix0 // CPB
        pib = pix0 % CPB
        obase = b * B_STRIDE + pib
        for c in range(16):
            pltpu.async_copy(
                out_v.at[pr, pl.ds(c * P, P)],
                out_hbm.at[pl.ds(obase + c * CH_STRIDE, P)], osem[pr])

    fire_uv(0, 0)
    fire_uv(1, 1)
    compute_block(0, 0)

    def pipe_body(t, _):
        blk = 2 * t
        compute_block(blk + 1, 1)
        finish_block(blk, 0)

        @pl.when(blk + 2 < nblk)
        def _():
            compute_block(blk + 2, 0)

        finish_block(blk + 1, 1)
        return 0

    lax.fori_loop(0, nblk // 2, pipe_body, 0, unroll=False)

    for pr in range(2):
        pltpu.make_async_copy(
            u_hbm.at[pl.ds(0, 16 * P)], out_v.at[pr], osem[pr]).wait()


@jax.jit
def _run(u, v, m0, m1, m2, m3):
    mesh = plsc.VectorSubcoreMesh(core_axis_name="c", subcore_axis_name="s")
    f = pl.kernel(
        _sc_body,
        out_type=(jax.ShapeDtypeStruct((NPIX * 16,), jnp.float32),
                  jax.ShapeDtypeStruct((TABV, 16), jnp.int32)),
        mesh=mesh,
        compiler_params=pltpu.CompilerParams(
            needs_layout_passes=False, use_tc_tiling_on_sc=False),
        scratch_types=[
            pltpu.VMEM((2, P), jnp.float32),
            pltpu.VMEM((2, P), jnp.float32),
            pltpu.VMEM((2, NROW), jnp.int32),
            pltpu.VMEM((2, 2 * NK * P), jnp.float32),
            pltpu.VMEM((2, NROW, 16), jnp.int32),
            pltpu.VMEM((2, 16 * P), jnp.float32),
            pltpu.VMEM((2, 16, T + 16), jnp.float32),
            pltpu.VMEM((2, T, 16), jnp.int32),
            [pltpu.SemaphoreType.DMA, pltpu.SemaphoreType.DMA],
            [pltpu.SemaphoreType.DMA, pltpu.SemaphoreType.DMA],
            [pltpu.SemaphoreType.DMA, pltpu.SemaphoreType.DMA],
            [pltpu.SemaphoreType.DMA, pltpu.SemaphoreType.DMA],
            [pltpu.SemaphoreType.DMA, pltpu.SemaphoreType.DMA],
        ],
    )
    out, _ = f(u, v, m0, m1, m2, m3)
    return out


def kernel(input, mipmap_0, mipmap_1, mipmap_2, mipmap_3):
    u = input[..., 0].reshape(-1)
    v = input[..., 1].reshape(-1)
    planes = [m[0].reshape(DEPTH, -1)
              for m in (mipmap_0, mipmap_1, mipmap_2, mipmap_3)]
    out = _run(u, v, *planes)
    return out.reshape(4, DEPTH, 512, 512)


# speedup vs baseline: 2.2437x; 1.0038x over previous
import jax
import jax.numpy as jnp
from jax import lax
from jax.experimental import pallas as pl
from jax.experimental.pallas import tpu as pltpu
from jax.experimental.pallas import tpu_sc as plsc

SIZE = 1024
DEPTH = 16
NPIX = 4 * 512 * 512
LEVEL_W = [SIZE >> l for l in range(4)]
LEVEL_BASE = [0, 1048576, 1310720, 1376256]
TABV = 1392640

P = 256
G = P // 16
NK = 8
NROW = NK * P
CHUNK = 128
NCH = NROW // CHUNK
CPB = 262144
CH_STRIDE = 262144
B_STRIDE = 16 * 262144
T = 512


def _sc_body(u_hbm, v_hbm, m0_hbm, m1_hbm, m2_hbm, m3_hbm,
             out_hbm, tab_hbm,
             u_v, v_v, idx_v, w_v, rows_v, out_v, cb_v, row_v,
             usem, gsem, osem, csem, tsem):
    info = plsc.get_sparse_core_info()
    nc = info.num_cores
    cid = lax.axis_index("c")
    sid = lax.axis_index("s")
    wid = sid * nc + cid
    npix_w = NPIX // (nc * info.num_subcores)
    nblk = npix_w // P
    pixw0 = wid * npix_w

    iota = lax.iota(jnp.int32, 16)

    def fire_reads(m_hbm, start, par):
        for ch in range(16):
            pltpu.async_copy(m_hbm.at[ch, pl.ds(start, T + 8)],
                             cb_v.at[par, ch, pl.ds(0, T + 8)], csem[par])

    def drain_reads(m_hbm, par):
        for ch in range(16):
            pltpu.make_async_copy(m_hbm.at[ch, pl.ds(0, T + 8)],
                                  cb_v.at[par, ch, pl.ds(0, T + 8)],
                                  csem[par]).wait()

    for l, m_hbm in enumerate((m0_hbm, m1_hbm, m2_hbm, m3_hbm)):
        hw = LEVEL_W[l] * LEVEL_W[l]
        per_tile = hw // 16
        nck = per_tile // T
        tile0 = sid * per_tile
        lvl_base = LEVEL_BASE[l]

        def win_start(ck, nck=nck, tile0=tile0):
            is_final = jnp.logical_and(ck == nck - 1, sid == 15)
            off = jnp.where(is_final, 8, 0)
            return tile0 + ck * T - off, off

        def do_chunk(ck, par, l=l, m_hbm=m_hbm, nck=nck, tile0=tile0,
                     lvl_base=lvl_base):
            t0 = tile0 + ck * T
            _, off = win_start(ck)
            drain_reads(m_hbm, par)

            @pl.when(ck >= 2)
            def _():
                pltpu.make_async_copy(
                    tab_hbm.at[pl.ds(0, T)], row_v.at[par],
                    tsem[par]).wait()

            def grp_body(g, _):
                r0 = g * 16 + off
                pos = g * 16 + iota
                for wd in range(8):
                    a0 = cb_v[par, 2 * wd, pl.ds(r0, 16)]
                    a1 = cb_v[par, 2 * wd + 1, pl.ds(r0, 16)]
                    wa = plsc.bitcast(
                        plsc.pack(a0, a1, format=plsc.PackFormat.INTERLEAVED),
                        jnp.int32)
                    b0 = cb_v[par, 2 * wd, pl.ds(r0 + 1, 16)]
                    b1 = cb_v[par, 2 * wd + 1, pl.ds(r0 + 1, 16)]
                    wb = plsc.bitcast(
                        plsc.pack(b0, b1, format=plsc.PackFormat.INTERLEAVED),
                        jnp.int32)
                    plsc.store_scatter(
                        row_v, [jnp.broadcast_to(par, (16,)), pos,
                                jnp.full((16,), wd, jnp.int32)], wa)
                    plsc.store_scatter(
                        row_v, [jnp.broadcast_to(par, (16,)), pos,
                                jnp.full((16,), 8 + wd, jnp.int32)], wb)
                return 0

            lax.fori_loop(0, T // 16, grp_body, 0, unroll=False)
            pltpu.async_copy(row_v.at[par],
                             tab_hbm.at[pl.ds(lvl_base + t0, T)], tsem[par])

            @pl.when(ck + 2 < nck)
            def _():
                s2, _o = win_start(ck + 2)
                fire_reads(m_hbm, s2, par)

        s0, _ = win_start(0)
        fire_reads(m_hbm, s0, 0)
        s1, _ = win_start(1)
        fire_reads(m_hbm, s1, 1)

        def pair_body(t, _):
            do_chunk(2 * t, 0)
            do_chunk(2 * t + 1, 1)
            return 0

        lax.fori_loop(0, nck // 2, pair_body, 0, unroll=False)
        for par in range(2):
            pltpu.make_async_copy(
                tab_hbm.at[pl.ds(0, T)], row_v.at[par],
                tsem[par]).wait()

    plsc.subcore_barrier()

    def fire_uv(blk, pr):
        pix0 = pixw0 + blk * P
        pltpu.async_copy(u_hbm.at[pl.ds(pix0, P)], u_v.at[pr], usem[pr])
        pltpu.async_copy(v_hbm.at[pl.ds(pix0, P)], v_v.at[pr], usem[pr])

    def compute_block(blk, pr):
        pltpu.make_async_copy(u_hbm.at[pl.ds(0, P)], u_v.at[pr], usem[pr]).wait()
        pltpu.make_async_copy(v_hbm.at[pl.ds(0, P)], v_v.at[pr], usem[pr]).wait()

        def idx_body(g, _):
            p0 = g * 16
            ux = u_v[pr, pl.ds(p0, 16)]
            uy = v_v[pr, pl.ds(p0, 16)]
            gx = 2.0 * ux - 1.0
            gy = 2.0 * uy - 1.0
            for l in range(4):
                w = LEVEL_W[l]
                ix = ((gx + 1.0) * w - 1.0) / 2.0
                iy = ((gy + 1.0) * w - 1.0) / 2.0
                tx = ix.astype(jnp.int32)
                x0 = tx - jnp.where(tx.astype(jnp.float32) > ix, 1, 0)
                ty = iy.astype(jnp.int32)
                y0 = ty - jnp.where(ty.astype(jnp.float32) > iy, 1, 0)
                wx1 = ix - x0.astype(jnp.float32)
                wx0 = 1.0 - wx1
                wy1 = iy - y0.astype(jnp.float32)
                wy0 = 1.0 - wy1
                is_l = x0 < 0
                is_r = x0 > w - 2
                sa = jnp.where(is_l, wx1, jnp.where(is_r, 0.0, wx0))
                sb = jnp.where(is_l, 0.0, jnp.where(is_r, wx0, wx1))
                ay0 = wy0 * jnp.where(y0 >= 0, 1.0, 0.0)
                ay1 = wy1 * jnp.where(y0 <= w - 2, 1.0, 0.0)
                xs = jnp.minimum(jnp.maximum(x0, 0), w - 2)
                yc0 = jnp.maximum(y0, 0)
                yc1 = jnp.minimum(y0 + 1, w - 1)
                base = LEVEL_BASE[l]
                for yr, (yc, ay) in enumerate(((yc0, ay0), (yc1, ay1))):
                    k = l * 2 + yr
                    idx_v[pr, pl.ds(k * P + p0, 16)] = yc * w + (base + xs)
                    w_v[pr, pl.ds((2 * k) * P + p0, 16)] = sa * ay
                    w_v[pr, pl.ds((2 * k + 1) * P + p0, 16)] = sb * ay
            return 0

        lax.fori_loop(0, G, idx_body, 0, unroll=False)
        for j in range(NCH):
            pltpu.async_copy(
                tab_hbm.at[idx_v.at[pr, pl.ds(j * CHUNK, CHUNK)]],
                rows_v.at[pr, pl.ds(j * CHUNK, CHUNK)], gsem[pr])

        @pl.when(blk + 2 < nblk)
        def _():
            fire_uv(blk + 2, pr)

    def finish_block(blk, pr):
        @pl.when(blk >= 2)
        def _():
            pltpu.make_async_copy(
                u_hbm.at[pl.ds(0, 16 * P)], out_v.at[pr], osem[pr]).wait()

        pltpu.make_async_copy(
            tab_hbm.at[pl.ds(0, NROW)], rows_v.at[pr], gsem[pr]).wait()

        mask_hi = jnp.full((16,), -65536, jnp.int32)

        def acc_body(g, _):
            p0 = g * 16
            accs = [jnp.zeros((16,), jnp.float32) for _ in range(16)]
            prv = jnp.broadcast_to(pr, (16,))
            for k in range(NK):
                wa = w_v[pr, pl.ds((2 * k) * P + p0, 16)]
                wb = w_v[pr, pl.ds((2 * k + 1) * P + p0, 16)]
                rid = k * P + p0 + iota
                for wd in range(8):
                    for half, wv in ((0, wa), (1, wb)):
                        vec = plsc.load_gather(
                            rows_v, [prv, rid,
                                     jnp.full((16,), 8 * half + wd, jnp.int32)])
                        ch_even = plsc.bitcast(vec << 16, jnp.float32)
                        ch_odd = plsc.bitcast(vec & mask_hi, jnp.float32)
                        accs[2 * wd] = accs[2 * wd] + wv * ch_even
                        accs[2 * wd + 1] = accs[2 * wd + 1] + wv * ch_odd
            for c in range(16):
                out_v[pr, pl.ds(c * P + p0, 16)] = accs[c]
            return 0

        lax.fori_loop(0, G, acc_body, 0, unroll=False)

        pix0 = pixw0 + blk * P
        b = pix0 // CPB
        pib = pix0 % CPB
        obase = b * B_STRIDE + pib
        for c in range(16):
            pltpu.async_copy(
                out_v.at[pr, pl.ds(c * P, P)],
                out_hbm.at[pl.ds(obase + c * CH_STRIDE, P)], osem[pr])

    fire_uv(0, 0)
    fire_uv(1, 1)
    compute_block(0, 0)

    def pipe_body(t, _):
        blk = 2 * t
        compute_block(blk + 1, 1)
        finish_block(blk, 0)

        @pl.when(blk + 2 < nblk)
        def _():
            compute_block(blk + 2, 0)

        finish_block(blk + 1, 1)
        return 0

    lax.fori_loop(0, nblk // 2, pipe_body, 0, unroll=False)

    for pr in range(2):
        pltpu.make_async_copy(
            u_hbm.at[pl.ds(0, 16 * P)], out_v.at[pr], osem[pr]).wait()


@jax.jit
def _run(u, v, m0, m1, m2, m3):
    mesh = plsc.VectorSubcoreMesh(core_axis_name="c", subcore_axis_name="s")
    f = pl.kernel(
        _sc_body,
        out_type=(jax.ShapeDtypeStruct((NPIX * 16,), jnp.float32),
                  jax.ShapeDtypeStruct((TABV, 16), jnp.int32)),
        mesh=mesh,
        compiler_params=pltpu.CompilerParams(
            needs_layout_passes=False, use_tc_tiling_on_sc=False),
        scratch_types=[
            pltpu.VMEM((2, P), jnp.float32),
            pltpu.VMEM((2, P), jnp.float32),
            pltpu.VMEM((2, NROW), jnp.int32),
            pltpu.VMEM((2, 2 * NK * P), jnp.float32),
            pltpu.VMEM((2, NROW, 16), jnp.int32),
            pltpu.VMEM((2, 16 * P), jnp.float32),
            pltpu.VMEM((2, 16, T + 16), jnp.float32),
            pltpu.VMEM((2, T, 16), jnp.int32),
            [pltpu.SemaphoreType.DMA, pltpu.SemaphoreType.DMA],
            [pltpu.SemaphoreType.DMA, pltpu.SemaphoreType.DMA],
            [pltpu.SemaphoreType.DMA, pltpu.SemaphoreType.DMA],
            [pltpu.SemaphoreType.DMA, pltpu.SemaphoreType.DMA],
            [pltpu.SemaphoreType.DMA, pltpu.SemaphoreType.DMA],
        ],
    )
    out, _ = f(u, v, m0, m1, m2, m3)
    return out


def kernel(input, mipmap_0, mipmap_1, mipmap_2, mipmap_3):
    u = input[..., 0].reshape(-1)
    v = input[..., 1].reshape(-1)
    planes = [m[0].reshape(DEPTH, -1)
              for m in (mipmap_0, mipmap_1, mipmap_2, mipmap_3)]
    out = _run(u, v, *planes)
    return out.reshape(4, DEPTH, 512, 512)
